# baseline (speedup 1.0000x reference)
import functools

import numpy as np
import jax
import jax.numpy as jnp
from jax.experimental import pallas as pl

H = 8
C = 40
SKC = 320
ENDC = 640
ODIM = 12
DIL = [1, 2, 1, 2, 1, 2, 1, 2]
NP = 208
NB = 8
INVBN = 1.0 / (1.0 + 1e-5) ** 0.5
F32 = jnp.float32
_DG = jax.lax.dot_general


def _dgt(a, b):
    return _DG(a, b, (((1,), (1,)), ((), ())), preferred_element_type=F32)


BF16 = jnp.bfloat16


def _dgtb(a, b):
    return _DG(a.astype(BF16), b.astype(BF16), (((1,), (1,)), ((), ())),
               preferred_element_type=F32)


def _ii(shape, dim):
    return jax.lax.broadcasted_iota(jnp.int32, shape, dim)


def _stem_k(x0_ref, x1_ref, sw_ref, cw_ref, bs_ref, bc_ref, out_ref, *, t):
    mask = (_ii((t, C * t), 0) == _ii((t, C * t), 1) % t).astype(F32)
    rcs = (_ii((C * t, C), 0) // t == _ii((C * t, C), 1)).astype(F32)
    ssw = mask * _dgt(sw_ref[...], rcs)
    scw = mask * _dgt(cw_ref[...], rcs)
    a = jnp.dot(x0_ref[...], ssw, preferred_element_type=F32) + _dgt(bs_ref[...], rcs)
    b = jnp.dot(x1_ref[...], scw, preferred_element_type=F32) + _dgt(bc_ref[...], rcs)
    out_ref[...] = a + jnp.where(b >= 0, b, 0.01 * b)


def _mask_k(src_ref, dst_ref, out_ref):
    s = src_ref[...]
    d = dst_ref[...]
    iota = jax.lax.broadcasted_iota(jnp.int32, (s.shape[0], NP), 1)
    sh = (iota == s).astype(F32)
    dh = (iota == d).astype(F32)
    c = _DG(dh, sh, (((0,), (0,)), ((), ())),
            preferred_element_type=F32)
    out_ref[...] = jnp.where(c > 0.5, jnp.log(c), -1e30)


def _gat2(hg, w2d_ref, ala_ref, ara_ref, lcnt, dout):
    w2d = w2d_ref[...]
    hd = H * dout
    blk2 = (_ii((2 * H, hd), 1) // dout ==
            _ii((2 * H, hd), 0) % H).astype(F32)
    alar = jnp.concatenate([ala_ref[...], ara_ref[...]], axis=0)
    m2 = jnp.tile(alar, (1, H)) * blk2
    vlr = jnp.dot(m2, w2d, preferred_element_type=F32)
    elT = _dgt(vlr[:H], hg)
    err = _dgt(hg, vlr[H:])
    accs = [jnp.zeros((NP, dout), F32) for _ in range(NB)]
    hgb = hg.astype(BF16)
    for h in range(H):
        feat = _DG(hgb, w2d[h * dout:(h + 1) * dout, :].astype(BF16),
                   (((1,), (1,)), ((), ())),
                   preferred_element_type=F32).astype(BF16)
        for b in range(NB):
            elrow = elT[h:h + 1, b * NP:(b + 1) * NP]
            ercol = err[b * NP:(b + 1) * NP, h:h + 1]
            zm = ercol + jnp.max(elrow)
            m = jnp.maximum(zm, 0.2 * zm)
            e = ercol + elrow
            e = jnp.maximum(e, 0.2 * e)
            sx = jnp.exp(e - m + lcnt)
            ss = jnp.sum(sx, axis=1, keepdims=True)
            rs = 1.0 / jnp.where(ss > 0, ss, 1.0)
            num = jnp.dot(sx.astype(BF16), feat[b * NP:(b + 1) * NP, :],
                          preferred_element_type=F32)
            rst = num * rs
            accs[b] = accs[b] + (jnp.maximum(rst, 0.0) +
                                 jnp.exp(jnp.minimum(rst, 0.0)) - 1.0)
    return jnp.concatenate(accs, axis=0) * (1.0 / H)


def _layer_body(h_ref, res_ref, wf_ref, wg_ref, bf_ref, bg_ref,
                wa_ref, ala_ref, ara_ref, wb_ref, alb_ref, arb_ref, lcnt_ref,
                out_ref, hlast_ref, *, tcur, tout, di, last):
    rr = (_ii((C * tcur, C), 0) // tcur == _ii((C * tcur, C), 1)).astype(F32)
    rc = (_ii((C * tout, C), 0) // tout == _ii((C * tout, C), 1)).astype(F32)
    sidx = _ii((C * tcur, C * tout), 0) % tcur
    tidx = _ii((C * tcur, C * tout), 1) % tout
    m0 = (sidx == tidx).astype(F32)
    m1 = (sidx == tidx + di).astype(F32)
    hv = h_ref[...]
    wf0, wf1 = wf_ref[0], wf_ref[1]
    wg0, wg1 = wg_ref[0], wg_ref[1]
    wfb = _dgt(_dgt(rr, wf0), rc) * m0 + _dgt(_dgt(rr, wf1), rc) * m1
    wgb = _dgt(_dgt(rr, wg0), rc) * m0 + _dgt(_dgt(rr, wg1), rc) * m1
    bft = _dgt(bf_ref[...], rc)
    bgt = _dgt(bg_ref[...], rc)
    hvb = hv.astype(BF16)
    f = jnp.tanh(jnp.dot(hvb, wfb.astype(BF16), preferred_element_type=F32)
                 + bft)
    g = jax.nn.sigmoid(jnp.dot(hvb, wgb.astype(BF16), preferred_element_type=F32)
                       + bgt)
    hn = f * g
    gsel = rc * (_ii((C * tout, C), 0) % tout == tout - 1).astype(F32)
    hlast_ref[...] = jnp.dot(hn, gsel, preferred_element_type=F32)
    if last:
        out_ref[...] = hn
        return
    lcnt = lcnt_ref[...]
    d = C * tout
    hga = _gat2(hn, wa_ref, ala_ref, ara_ref, lcnt, d)
    hgb = _gat2(hga, wb_ref, alb_ref, arb_ref, lcnt, d)
    out_ref[...] = (hgb + hn + res_ref[...]) * INVBN


def _head_k(hl_ref, wsk_ref, bsk_ref, w1_ref, b1_ref, w2_ref, b2_ref, out_ref):
    skip = _dgtb(hl_ref[...], wsk_ref[...]) + bsk_ref[...]
    o = jnp.maximum(skip, 0.0)
    o = jnp.maximum(_dgtb(o, w1_ref[...]) + b1_ref[...], 0.0)
    out_ref[...] = _dgtb(o, w2_ref[...]) + b2_ref[...]


def _call(body, outs, *args):
    return pl.pallas_call(
        body,
        in_specs=[pl.BlockSpec(a.shape, lambda *_, _n=a.ndim: (0,) * _n)
                  for a in args],
        out_specs=jax.tree.map(
            lambda s: pl.BlockSpec(s.shape, lambda *_: (0,) * len(s.shape)), outs),
        out_shape=outs,
    )(*args)


def kernel(x, params, edge_index):
    p = params
    B, _, N, T = x.shape
    BN = B * NP

    xt = jnp.transpose(x, (0, 2, 1, 3))
    xt = jnp.pad(xt, ((0, 0), (0, NP - N), (0, 0), (0, 0)))
    x0 = xt[:, :, 0, :].reshape(BN, T)
    x1 = xt[:, :, 1, :].reshape(BN, T)

    h = _call(functools.partial(_stem_k, t=T),
              jax.ShapeDtypeStruct((BN, C * T), F32),
              x0, x1, p['start_W'].reshape(1, C), p['cat_W'].reshape(1, C),
              p['start_b'][None, :], p['cat_b'][None, :])

    E = edge_index.shape[1]
    ep = ((E + 7) // 8) * 8
    pad = jnp.full((ep - E,), 255, jnp.int32)
    srcp = jnp.concatenate([edge_index[0], pad])[:, None]
    dstp = jnp.concatenate([edge_index[1], pad])[:, None]
    lcnt = _call(_mask_k, jax.ShapeDtypeStruct((NP, NP), F32), srcp, dstp)

    tcur = T
    hlasts = []
    for i in range(len(DIL)):
        di = DIL[i]
        tout = tcur - di
        d = C * tout
        last = i == len(DIL) - 1
        wf2 = p['filt_W'][i][:, :, 0, :].transpose(2, 0, 1)
        wg2 = p['gate_W'][i][:, :, 0, :].transpose(2, 0, 1)
        body = functools.partial(_layer_body, tcur=tcur, tout=tout, di=di,
                                 last=last)
        outs = [jax.ShapeDtypeStruct((BN, d), F32),
                jax.ShapeDtypeStruct((BN, C), F32)]
        if last:
            z = jnp.zeros((1, 1), F32)
            h, hlast = _call(body, outs, h, z, wf2, wg2,
                             p['filt_b'][i][None, :], p['gate_b'][i][None, :],
                             z, z, z, z, z, z, z)
        else:
            res = h.reshape(BN, C, tcur)[:, :, tcur - tout:].reshape(BN, d)
            h, hlast = _call(
                body, outs, h, res, wf2, wg2,
                p['filt_b'][i][None, :], p['gate_b'][i][None, :],
                p['g%da_fcW' % i], p['g%da_al' % i], p['g%da_ar' % i],
                p['g%db_fcW' % i], p['g%db_al' % i], p['g%db_ar' % i], lcnt)
        hlasts.append(hlast)
        tcur = tout

    hl = jnp.concatenate(hlasts, axis=1)
    wskc = jnp.concatenate([p['skip_W'][i][:, :, 0, 0] for i in range(len(DIL))],
                           axis=1)
    bsk = jnp.sum(p['skip_b'], axis=0)[None, :]
    out2d = _call(_head_k, jax.ShapeDtypeStruct((BN, ODIM), F32),
                  hl, wskc, bsk, p['end1_W'][:, :, 0, 0], p['end1_b'][None, :],
                  p['end2_W'][:, :, 0, 0], p['end2_b'][None, :])

    out = out2d.reshape(B, NP, ODIM)[:, :N, :].transpose(0, 2, 1)[:, :, :, None]
    return out


# speedup vs baseline: 1.0809x; 1.0809x over previous
import functools

import numpy as np
import jax
import jax.numpy as jnp
from jax.experimental import pallas as pl

H = 8
C = 40
SKC = 320
ENDC = 640
ODIM = 12
DIL = [1, 2, 1, 2, 1, 2, 1, 2]
NP = 208
NB = 8
INVBN = 1.0 / (1.0 + 1e-5) ** 0.5
F32 = jnp.float32
_DG = jax.lax.dot_general


def _dgt(a, b):
    return _DG(a, b, (((1,), (1,)), ((), ())), preferred_element_type=F32)


def _ii(shape, dim):
    return jax.lax.broadcasted_iota(jnp.int32, shape, dim)


def _stem_k(x0_ref, x1_ref, sw_ref, cw_ref, bs_ref, bc_ref, out_ref, *, t):
    mask = (_ii((t, C * t), 0) == _ii((t, C * t), 1) % t).astype(F32)
    rcs = (_ii((C * t, C), 0) // t == _ii((C * t, C), 1)).astype(F32)
    ssw = mask * _dgt(sw_ref[...], rcs)
    scw = mask * _dgt(cw_ref[...], rcs)
    a = jnp.dot(x0_ref[...], ssw, preferred_element_type=F32) + _dgt(bs_ref[...], rcs)
    b = jnp.dot(x1_ref[...], scw, preferred_element_type=F32) + _dgt(bc_ref[...], rcs)
    out_ref[...] = a + jnp.where(b >= 0, b, 0.01 * b)


def _mask_k(src_ref, dst_ref, out_ref):
    s = src_ref[...]
    d = dst_ref[...]
    iota = jax.lax.broadcasted_iota(jnp.int32, (s.shape[0], NP), 1)
    sh = (iota == s).astype(F32)
    dh = (iota == d).astype(F32)
    c = _DG(dh, sh, (((0,), (0,)), ((), ())),
            preferred_element_type=F32)
    out_ref[...] = jnp.where(c > 0.5, jnp.log(c), -1e30)


def _gat2(hg, w2d_ref, ala_ref, ara_ref, lcnt, dout):
    w2d = w2d_ref[...]
    hd = H * dout
    blk2 = (_ii((2 * H, hd), 1) // dout ==
            _ii((2 * H, hd), 0) % H).astype(F32)
    alar = jnp.concatenate([ala_ref[...], ara_ref[...]], axis=0)
    m2 = jnp.tile(alar, (1, H)) * blk2
    vlr = jnp.dot(m2, w2d, preferred_element_type=F32)
    elT = _dgt(vlr[:H], hg)
    err = _dgt(hg, vlr[H:])
    accs = [jnp.zeros((NP, dout), F32) for _ in range(NB)]
    for h in range(H):
        feat = _dgt(hg, w2d[h * dout:(h + 1) * dout, :])
        for b in range(NB):
            elrow = elT[h:h + 1, b * NP:(b + 1) * NP]
            ercol = err[b * NP:(b + 1) * NP, h:h + 1]
            zm = ercol + jnp.max(elrow)
            m = jnp.maximum(zm, 0.2 * zm)
            e = ercol + elrow
            e = jnp.maximum(e, 0.2 * e)
            sx = jnp.exp(e - m + lcnt)
            ss = jnp.sum(sx, axis=1, keepdims=True)
            rs = 1.0 / jnp.where(ss > 0, ss, 1.0)
            num = jnp.dot(sx, feat[b * NP:(b + 1) * NP, :],
                          preferred_element_type=F32)
            rst = num * rs
            accs[b] = accs[b] + (jnp.maximum(rst, 0.0) +
                                 jnp.exp(jnp.minimum(rst, 0.0)) - 1.0)
    return jnp.concatenate(accs, axis=0) * (1.0 / H)


def _layer_body(h_ref, res_ref, wf_ref, wg_ref, bf_ref, bg_ref,
                wa_ref, ala_ref, ara_ref, wb_ref, alb_ref, arb_ref, lcnt_ref,
                out_ref, hlast_ref, *, tcur, tout, di, last):
    rr = (_ii((C * tcur, C), 0) // tcur == _ii((C * tcur, C), 1)).astype(F32)
    rc = (_ii((C * tout, C), 0) // tout == _ii((C * tout, C), 1)).astype(F32)
    sidx = _ii((C * tcur, C * tout), 0) % tcur
    tidx = _ii((C * tcur, C * tout), 1) % tout
    m0 = (sidx == tidx).astype(F32)
    m1 = (sidx == tidx + di).astype(F32)
    hv = h_ref[...]
    wf0, wf1 = wf_ref[0], wf_ref[1]
    wg0, wg1 = wg_ref[0], wg_ref[1]
    wfb = _dgt(_dgt(rr, wf0), rc) * m0 + _dgt(_dgt(rr, wf1), rc) * m1
    wgb = _dgt(_dgt(rr, wg0), rc) * m0 + _dgt(_dgt(rr, wg1), rc) * m1
    bft = _dgt(bf_ref[...], rc)
    bgt = _dgt(bg_ref[...], rc)
    f = jnp.tanh(jnp.dot(hv, wfb, preferred_element_type=F32) + bft)
    g = jax.nn.sigmoid(jnp.dot(hv, wgb, preferred_element_type=F32) + bgt)
    hn = f * g
    gsel = rc * (_ii((C * tout, C), 0) % tout == tout - 1).astype(F32)
    hlast_ref[...] = jnp.dot(hn, gsel, preferred_element_type=F32)
    if last:
        out_ref[...] = hn
        return
    lcnt = lcnt_ref[...]
    d = C * tout
    hga = _gat2(hn, wa_ref, ala_ref, ara_ref, lcnt, d)
    hgb = _gat2(hga, wb_ref, alb_ref, arb_ref, lcnt, d)
    out_ref[...] = (hgb + hn + res_ref[...]) * INVBN


def _head_k(hl_ref, wsk_ref, bsk_ref, w1_ref, b1_ref, w2_ref, b2_ref, out_ref):
    skip = _dgt(hl_ref[...], wsk_ref[...]) + bsk_ref[...]
    o = jnp.maximum(skip, 0.0)
    o = jnp.maximum(_dgt(o, w1_ref[...]) + b1_ref[...], 0.0)
    out_ref[...] = _dgt(o, w2_ref[...]) + b2_ref[...]


def _call(body, outs, *args):
    return pl.pallas_call(
        body,
        in_specs=[pl.BlockSpec(a.shape, lambda *_, _n=a.ndim: (0,) * _n)
                  for a in args],
        out_specs=jax.tree.map(
            lambda s: pl.BlockSpec(s.shape, lambda *_: (0,) * len(s.shape)), outs),
        out_shape=outs,
    )(*args)


def kernel(x, params, edge_index):
    p = params
    B, _, N, T = x.shape
    BN = B * NP

    xt = jnp.transpose(x, (0, 2, 1, 3))
    xt = jnp.pad(xt, ((0, 0), (0, NP - N), (0, 0), (0, 0)))
    x0 = xt[:, :, 0, :].reshape(BN, T)
    x1 = xt[:, :, 1, :].reshape(BN, T)

    h = _call(functools.partial(_stem_k, t=T),
              jax.ShapeDtypeStruct((BN, C * T), F32),
              x0, x1, p['start_W'].reshape(1, C), p['cat_W'].reshape(1, C),
              p['start_b'][None, :], p['cat_b'][None, :])

    E = edge_index.shape[1]
    ep = ((E + 7) // 8) * 8
    pad = jnp.full((ep - E,), 255, jnp.int32)
    srcp = jnp.concatenate([edge_index[0], pad])[:, None]
    dstp = jnp.concatenate([edge_index[1], pad])[:, None]
    lcnt = _call(_mask_k, jax.ShapeDtypeStruct((NP, NP), F32), srcp, dstp)

    tcur = T
    hlasts = []
    for i in range(len(DIL)):
        di = DIL[i]
        tout = tcur - di
        d = C * tout
        last = i == len(DIL) - 1
        wf2 = p['filt_W'][i][:, :, 0, :].transpose(2, 0, 1)
        wg2 = p['gate_W'][i][:, :, 0, :].transpose(2, 0, 1)
        body = functools.partial(_layer_body, tcur=tcur, tout=tout, di=di,
                                 last=last)
        outs = [jax.ShapeDtypeStruct((BN, d), F32),
                jax.ShapeDtypeStruct((BN, C), F32)]
        if last:
            z = jnp.zeros((1, 1), F32)
            h, hlast = _call(body, outs, h, z, wf2, wg2,
                             p['filt_b'][i][None, :], p['gate_b'][i][None, :],
                             z, z, z, z, z, z, z)
        else:
            res = h.reshape(BN, C, tcur)[:, :, tcur - tout:].reshape(BN, d)
            h, hlast = _call(
                body, outs, h, res, wf2, wg2,
                p['filt_b'][i][None, :], p['gate_b'][i][None, :],
                p['g%da_fcW' % i], p['g%da_al' % i], p['g%da_ar' % i],
                p['g%db_fcW' % i], p['g%db_al' % i], p['g%db_ar' % i], lcnt)
        hlasts.append(hlast)
        tcur = tout

    hl = jnp.concatenate(hlasts, axis=1)
    wskc = jnp.concatenate([p['skip_W'][i][:, :, 0, 0] for i in range(len(DIL))],
                           axis=1)
    bsk = jnp.sum(p['skip_b'], axis=0)[None, :]
    out2d = _call(_head_k, jax.ShapeDtypeStruct((BN, ODIM), F32),
                  hl, wskc, bsk, p['end1_W'][:, :, 0, 0], p['end1_b'][None, :],
                  p['end2_W'][:, :, 0, 0], p['end2_b'][None, :])

    out = out2d.reshape(B, NP, ODIM)[:, :N, :].transpose(0, 2, 1)[:, :, :, None]
    return out


# speedup vs baseline: 1.4102x; 1.3046x over previous
import functools

import numpy as np
import jax
import jax.numpy as jnp
from jax.experimental import pallas as pl

H = 8
C = 40
SKC = 320
ENDC = 640
ODIM = 12
DIL = [1, 2, 1, 2, 1, 2, 1, 2]
NP = 208
NB = 8
INVBN = 1.0 / (1.0 + 1e-5) ** 0.5
F32 = jnp.float32
_DG = jax.lax.dot_general


def _dgt(a, b):
    return _DG(a, b, (((1,), (1,)), ((), ())), preferred_element_type=F32)


def _ii(shape, dim):
    return jax.lax.broadcasted_iota(jnp.int32, shape, dim)


def _stem_k(x0_ref, x1_ref, sw_ref, cw_ref, bs_ref, bc_ref, out_ref, *, t):
    mask = (_ii((t, C * t), 0) == _ii((t, C * t), 1) % t).astype(F32)
    rcs = (_ii((C * t, C), 0) // t == _ii((C * t, C), 1)).astype(F32)
    ssw = mask * _dgt(sw_ref[...], rcs)
    scw = mask * _dgt(cw_ref[...], rcs)
    a = jnp.dot(x0_ref[...], ssw, preferred_element_type=F32) + _dgt(bs_ref[...], rcs)
    b = jnp.dot(x1_ref[...], scw, preferred_element_type=F32) + _dgt(bc_ref[...], rcs)
    out_ref[...] = a + jnp.where(b >= 0, b, 0.01 * b)


def _mask_k(src_ref, dst_ref, out_ref):
    s = src_ref[...]
    d = dst_ref[...]
    iota = jax.lax.broadcasted_iota(jnp.int32, (s.shape[0], NP), 1)
    sh = (iota == s).astype(F32)
    dh = (iota == d).astype(F32)
    c = _DG(dh, sh, (((0,), (0,)), ((), ())),
            preferred_element_type=F32)
    lc = jnp.where(c > 0.5, jnp.log(c), -1e30)
    out_ref[...] = jnp.concatenate([lc] * NB, axis=0)


def _gat2(hg, w2d_ref, ala_ref, ara_ref, lcnt, dout):
    w2d = w2d_ref[...]
    hd = H * dout
    bn = NB * NP
    blk2 = (_ii((2 * H, hd), 1) // dout ==
            _ii((2 * H, hd), 0) % H).astype(F32)
    alar = jnp.concatenate([ala_ref[...], ara_ref[...]], axis=0)
    m2 = jnp.tile(alar, (1, H)) * blk2
    vlr = jnp.dot(m2, w2d, preferred_element_type=F32)
    elT = _dgt(vlr[:H], hg)
    err = _dgt(hg, vlr[H:])
    acc = jnp.zeros((bn, dout), F32)
    for h in range(H):
        el_block = jnp.concatenate(
            [jnp.broadcast_to(elT[h:h + 1, b * NP:(b + 1) * NP], (NP, NP))
             for b in range(NB)], axis=0)
        ercol = err[:, h:h + 1]
        zm = ercol + jnp.max(elT[h:h + 1, :])
        m = jnp.maximum(zm, 0.2 * zm)
        e = ercol + el_block
        e = jnp.maximum(e, 0.2 * e)
        sx = jnp.exp(e - m + lcnt)
        ss = jnp.sum(sx, axis=1, keepdims=True)
        rs = 1.0 / jnp.where(ss > 0, ss, 1.0)
        feat = _dgt(hg, w2d[h * dout:(h + 1) * dout, :])
        nums = [jnp.dot(sx[b * NP:(b + 1) * NP, :],
                        feat[b * NP:(b + 1) * NP, :],
                        preferred_element_type=F32) for b in range(NB)]
        rst = jnp.concatenate(nums, axis=0) * rs
        acc = acc + (jnp.maximum(rst, 0.0) +
                     jnp.exp(jnp.minimum(rst, 0.0)) - 1.0)
    return acc * (1.0 / H)


def _layer_body(h_ref, res_ref, wf_ref, wg_ref, bf_ref, bg_ref,
                wa_ref, ala_ref, ara_ref, wb_ref, alb_ref, arb_ref, lcnt_ref,
                out_ref, hlast_ref, *, tcur, tout, di, last):
    rr = (_ii((C * tcur, C), 0) // tcur == _ii((C * tcur, C), 1)).astype(F32)
    rc = (_ii((C * tout, C), 0) // tout == _ii((C * tout, C), 1)).astype(F32)
    sidx = _ii((C * tcur, C * tout), 0) % tcur
    tidx = _ii((C * tcur, C * tout), 1) % tout
    m0 = (sidx == tidx).astype(F32)
    m1 = (sidx == tidx + di).astype(F32)
    hv = h_ref[...]
    wf0, wf1 = wf_ref[0], wf_ref[1]
    wg0, wg1 = wg_ref[0], wg_ref[1]
    wfb = _dgt(_dgt(rr, wf0), rc) * m0 + _dgt(_dgt(rr, wf1), rc) * m1
    wgb = _dgt(_dgt(rr, wg0), rc) * m0 + _dgt(_dgt(rr, wg1), rc) * m1
    bft = _dgt(bf_ref[...], rc)
    bgt = _dgt(bg_ref[...], rc)
    f = jnp.tanh(jnp.dot(hv, wfb, preferred_element_type=F32) + bft)
    g = jax.nn.sigmoid(jnp.dot(hv, wgb, preferred_element_type=F32) + bgt)
    hn = f * g
    gsel = rc * (_ii((C * tout, C), 0) % tout == tout - 1).astype(F32)
    hlast_ref[...] = jnp.dot(hn, gsel, preferred_element_type=F32)
    if last:
        out_ref[...] = hn
        return
    lcnt = lcnt_ref[...]
    d = C * tout
    hga = _gat2(hn, wa_ref, ala_ref, ara_ref, lcnt, d)
    hgb = _gat2(hga, wb_ref, alb_ref, arb_ref, lcnt, d)
    out_ref[...] = (hgb + hn + res_ref[...]) * INVBN


def _head_k(hl_ref, wsk_ref, bsk_ref, w1_ref, b1_ref, w2_ref, b2_ref, out_ref):
    skip = _dgt(hl_ref[...], wsk_ref[...]) + bsk_ref[...]
    o = jnp.maximum(skip, 0.0)
    o = jnp.maximum(_dgt(o, w1_ref[...]) + b1_ref[...], 0.0)
    out_ref[...] = _dgt(o, w2_ref[...]) + b2_ref[...]


def _call(body, outs, *args):
    return pl.pallas_call(
        body,
        in_specs=[pl.BlockSpec(a.shape, lambda *_, _n=a.ndim: (0,) * _n)
                  for a in args],
        out_specs=jax.tree.map(
            lambda s: pl.BlockSpec(s.shape, lambda *_: (0,) * len(s.shape)), outs),
        out_shape=outs,
    )(*args)


def kernel(x, params, edge_index):
    p = params
    B, _, N, T = x.shape
    BN = B * NP

    xt = jnp.transpose(x, (0, 2, 1, 3))
    xt = jnp.pad(xt, ((0, 0), (0, NP - N), (0, 0), (0, 0)))
    x0 = xt[:, :, 0, :].reshape(BN, T)
    x1 = xt[:, :, 1, :].reshape(BN, T)

    h = _call(functools.partial(_stem_k, t=T),
              jax.ShapeDtypeStruct((BN, C * T), F32),
              x0, x1, p['start_W'].reshape(1, C), p['cat_W'].reshape(1, C),
              p['start_b'][None, :], p['cat_b'][None, :])

    E = edge_index.shape[1]
    ep = ((E + 7) // 8) * 8
    pad = jnp.full((ep - E,), 255, jnp.int32)
    srcp = jnp.concatenate([edge_index[0], pad])[:, None]
    dstp = jnp.concatenate([edge_index[1], pad])[:, None]
    lcnt = _call(_mask_k, jax.ShapeDtypeStruct((NB * NP, NP), F32), srcp, dstp)

    tcur = T
    hlasts = []
    for i in range(len(DIL)):
        di = DIL[i]
        tout = tcur - di
        d = C * tout
        last = i == len(DIL) - 1
        wf2 = p['filt_W'][i][:, :, 0, :].transpose(2, 0, 1)
        wg2 = p['gate_W'][i][:, :, 0, :].transpose(2, 0, 1)
        body = functools.partial(_layer_body, tcur=tcur, tout=tout, di=di,
                                 last=last)
        outs = [jax.ShapeDtypeStruct((BN, d), F32),
                jax.ShapeDtypeStruct((BN, C), F32)]
        if last:
            z = jnp.zeros((1, 1), F32)
            h, hlast = _call(body, outs, h, z, wf2, wg2,
                             p['filt_b'][i][None, :], p['gate_b'][i][None, :],
                             z, z, z, z, z, z, z)
        else:
            res = h.reshape(BN, C, tcur)[:, :, tcur - tout:].reshape(BN, d)
            h, hlast = _call(
                body, outs, h, res, wf2, wg2,
                p['filt_b'][i][None, :], p['gate_b'][i][None, :],
                p['g%da_fcW' % i], p['g%da_al' % i], p['g%da_ar' % i],
                p['g%db_fcW' % i], p['g%db_al' % i], p['g%db_ar' % i], lcnt)
        hlasts.append(hlast)
        tcur = tout

    hl = jnp.concatenate(hlasts, axis=1)
    wskc = jnp.concatenate([p['skip_W'][i][:, :, 0, 0] for i in range(len(DIL))],
                           axis=1)
    bsk = jnp.sum(p['skip_b'], axis=0)[None, :]
    out2d = _call(_head_k, jax.ShapeDtypeStruct((BN, ODIM), F32),
                  hl, wskc, bsk, p['end1_W'][:, :, 0, 0], p['end1_b'][None, :],
                  p['end2_W'][:, :, 0, 0], p['end2_b'][None, :])

    out = out2d.reshape(B, NP, ODIM)[:, :N, :].transpose(0, 2, 1)[:, :, :, None]
    return out
